# baseline (speedup 1.0000x reference)
import functools

import jax
import jax.numpy as jnp
from jax import lax
from jax.experimental import pallas as pl
from jax.experimental.pallas import tpu as pltpu
from jax.experimental.pallas import tpu_sc as plsc

_B = 16
_N = 10000
_F = 128
_DOUT = 64
_NMAT = 5
_E = 320000

_NW = 32
_RPT = 320
_NPAD = _NW * _RPT
_C = 256
_NCH = (_B * _F) // _C
_EPT = 12288
_BATCH = 64

_NBLK = 256


def _spmm_body(x_ref, dstl_ref, col_ref, val_ref, nb_ref, y_ref,
               nb_v, idx_v, dl_v, vv_v, gbuf_v, acc_v, sem):
    w = lax.axis_index("s") * 2 + lax.axis_index("c")
    pltpu.sync_copy(nb_ref, nb_v.at[pl.ds(0, _NW)])
    nb = nb_v[pl.ds(w, 16)][0]
    ebase = w * _EPT

    for ch in range(_NCH):
        chbase = ch * _NPAD

        def _zero_row(r, _):
            for k in range(_C // 16):
                acc_v[r, pl.ds(k * 16, 16)] = jnp.zeros((16,), jnp.float32)
            return 0
        lax.fori_loop(0, _RPT, _zero_row, 0)

        def _batch(b, _):
            off = ebase + b * _BATCH
            pltpu.sync_copy(col_ref.at[pl.ds(off, _BATCH)], idx_v)
            for t in range(_BATCH // 16):
                idx_v[pl.ds(t * 16, 16)] = idx_v[pl.ds(t * 16, 16)] + chbase
            pltpu.sync_copy(dstl_ref.at[pl.ds(off, _BATCH)], dl_v.at[pl.ds(0, _BATCH)])
            pltpu.sync_copy(val_ref.at[pl.ds(off, _BATCH)], vv_v.at[pl.ds(0, _BATCH)])
            pltpu.async_copy(x_ref.at[idx_v], gbuf_v, sem).wait()

            def _edge(j, _):
                dl = dl_v[pl.ds(j, 16)][0]
                v = vv_v[pl.ds(j, 16)][0]
                for k in range(_C // 16):
                    sl = pl.ds(k * 16, 16)
                    acc_v[dl, sl] = acc_v[dl, sl] + v * gbuf_v[j, sl]
                return 0
            lax.fori_loop(0, _BATCH, _edge, 0)
            return 0
        lax.fori_loop(0, nb, _batch, 0)

        pltpu.sync_copy(acc_v, y_ref.at[pl.ds(chbase + w * _RPT, _RPT)])


def _spmm_sc(x_flat, dstl, col, val, nb):
    mesh = plsc.VectorSubcoreMesh(core_axis_name="c", subcore_axis_name="s")
    f = pl.kernel(
        _spmm_body,
        out_type=jax.ShapeDtypeStruct((_NCH * _NPAD, _C), jnp.float32),
        mesh=mesh,
        scratch_types=[
            pltpu.VMEM((_NW + 16,), jnp.int32),
            pltpu.VMEM((_BATCH,), jnp.int32),
            pltpu.VMEM((_BATCH + 16,), jnp.int32),
            pltpu.VMEM((_BATCH + 16,), jnp.float32),
            pltpu.VMEM((_BATCH, _C), jnp.float32),
            pltpu.VMEM((_RPT, _C), jnp.float32),
            pltpu.SemaphoreType.DMA,
        ],
    )
    return f(x_flat, dstl, col, val, nb)


def _prep_edges(edge_index, values):
    dst = edge_index[0]
    src = edge_index[1]
    bucket = dst // _RPT
    perm = jnp.argsort(bucket, stable=True)
    dst_s = jnp.take(dst, perm)
    col_s = jnp.take(src, perm)
    val_s = jnp.take(values, perm)
    counts = jnp.bincount(bucket, length=_NW)
    starts = jnp.concatenate([jnp.zeros((1,), jnp.int32),
                              jnp.cumsum(counts).astype(jnp.int32)])
    k = jnp.arange(_EPT, dtype=jnp.int32)
    srcpos = jnp.minimum(starts[:-1, None] + k[None, :], _E - 1)
    valid = k[None, :] < counts[:, None]
    tile_row0 = (jnp.arange(_NW, dtype=jnp.int32) * _RPT)[:, None]
    dstl_p = jnp.where(valid, jnp.take(dst_s, srcpos) - tile_row0, 0)
    col_p = jnp.where(valid, jnp.take(col_s, srcpos), 0)
    val_p = jnp.where(valid, jnp.take(val_s, srcpos), 0.0)
    nb = ((counts + (_BATCH - 1)) // _BATCH).astype(jnp.int32)
    return (dstl_p.reshape(-1).astype(jnp.int32),
            col_p.reshape(-1).astype(jnp.int32),
            val_p.reshape(-1).astype(jnp.float32), nb)


def _combine_body(x0_ref, y1_ref, y2_ref, y3_ref, y4_ref, w_ref, b_ref, out_ref):
    refs = (x0_ref, y1_ref, y2_ref, y3_ref, y4_ref)
    for c in range(_NCH):
        xc = jnp.concatenate(
            [r[c].reshape(_NBLK * 2, _F) for r in refs], axis=1)
        acc = jnp.dot(xc, w_ref[...], preferred_element_type=jnp.float32)
        acc = acc + b_ref[...].reshape(1, _DOUT)
        out_ref[pl.ds(2 * c, 2)] = acc.reshape(_NBLK, 2, _DOUT).transpose(1, 0, 2)


def _combine(mats, w_all, biases):
    grid = (_NPAD // _NBLK,)
    in_specs = [pl.BlockSpec((_NCH, _NBLK, _C), lambda i: (0, i, 0))
                for _ in range(5)]
    in_specs.append(pl.BlockSpec((_NMAT * _F, _DOUT), lambda i: (0, 0)))
    in_specs.append(pl.BlockSpec((1, _DOUT), lambda i: (0, 0)))
    return pl.pallas_call(
        _combine_body,
        grid=grid,
        in_specs=in_specs,
        out_specs=pl.BlockSpec((_B, _NBLK, _DOUT), lambda i: (0, i, 0)),
        out_shape=jax.ShapeDtypeStruct((_B, _NPAD, _DOUT), jnp.float32),
    )(*[m.reshape(_NCH, _NPAD, _C) for m in mats], w_all,
      biases.reshape(1, _DOUT))


def kernel(inputs, state, edge_index1, values1, edge_index2, values2, weight, biases):
    x_in = inputs.reshape(_B, _N, -1)
    st = state.reshape(_B, _N, -1)
    xs = jnp.concatenate([x_in, st], axis=2)
    x_bi = jnp.transpose(xs, (1, 0, 2)).reshape(_N, _B * _F)
    x_ch = x_bi.reshape(_N, _NCH, _C).transpose(1, 0, 2)
    x0f = jnp.pad(x_ch, ((0, 0), (0, _NPAD - _N), (0, 0))).reshape(-1, _C)

    p1 = _prep_edges(edge_index1, values1)
    p2 = _prep_edges(edge_index2, values2)

    y1 = _spmm_sc(x0f, *p1)
    y2 = _spmm_sc(y1, *p1)
    y3 = _spmm_sc(x0f, *p2)
    y4 = _spmm_sc(y3, *p2)

    w = weight.reshape(_F, _NMAT, _DOUT)
    w_all = jnp.concatenate([
        w[:, 0] - w[:, 2] - w[:, 4],
        w[:, 1],
        2.0 * w[:, 2],
        w[:, 3],
        2.0 * w[:, 4],
    ], axis=0)

    out = _combine((x0f, y1, y2, y3, y4), w_all, biases)
    return out[:, :_N, :].reshape(_B, _N * _DOUT)
